# baseline (speedup 1.0000x reference)
import functools

import jax
import jax.numpy as jnp
from jax.experimental import pallas as pl

N = 2048
BM = 256
NB = N // BM
CUTOFF2 = 1.5 * 1.5


def _w_block(jb, PE, Pt):
    rows = slice(jb * BM, (jb + 1) * BM)
    Pr = PE[rows, 0:3]
    sqr = PE[rows, 4:5]
    sqc = Pt[3:4, :]
    d2 = sqr + sqc - 2.0 * (
        Pr[:, 0:1] * Pt[0:1, :]
        + Pr[:, 1:2] * Pt[1:2, :]
        + Pr[:, 2:3] * Pt[2:3, :]
    )
    row_ids = jax.lax.broadcasted_iota(jnp.int32, (BM, N), 0) + jb * BM
    col_ids = jax.lax.broadcasted_iota(jnp.int32, (BM, N), 1)
    mask = (d2 < CUTOFF2) & (row_ids != col_ids)
    W = jnp.where(mask, jax.lax.rsqrt(jnp.maximum(d2, 0.0) + 1e-12), 0.0)
    deg = jnp.sum(jnp.where(mask, 1.0, 0.0), axis=1, keepdims=True)
    invdeg = 1.0 / jnp.maximum(deg, 1.0)
    has = (deg > 0.0).astype(jnp.float32)
    return W, Pr, invdeg, has


def _body(pe_ref, pt_ref, ab_ref, wo_ref, bo_ref, out_ref):
    PE = pe_ref[:]
    Pt = pt_ref[:]
    a = ab_ref[0:1, :]
    b = ab_ref[1:2, :]

    uc_blocks = []
    aux = []
    for jb in range(NB):
        W, Pr, invdeg, has = _w_block(jb, PE, Pt)
        A = jnp.dot(W, PE[:, 0:4], preferred_element_type=jnp.float32)
        U = (Pr * A[:, 3:4] - A[:, 0:3]) * invdeg
        c = jnp.sum(U * Pr, axis=1, keepdims=True)
        uc_blocks.append(jnp.concatenate([U, c], axis=1))
        aux.append((invdeg, has))
    UC = jnp.concatenate(uc_blocks, axis=0)

    acc = jnp.zeros((1, 10), dtype=jnp.float32)
    for jb in range(NB):
        W, Pr, invdeg, has = _w_block(jb, PE, Pt)
        B = jnp.dot(W, UC, preferred_element_type=jnp.float32)
        t = (jnp.sum(Pr * B[:, 0:3], axis=1, keepdims=True)
             - B[:, 3:4]) * invdeg
        node2 = jax.nn.gelu(has * a + t * b)
        acc = acc + jnp.sum(node2, axis=0, keepdims=True)

    pooled = acc * (1.0 / N)
    logits = jnp.dot(pooled, wo_ref[:],
                     preferred_element_type=jnp.float32) + bo_ref[:]
    out_ref[:] = jax.nn.softmax(logits, axis=-1)


@functools.partial(jax.jit, static_argnames=())
def kernel(positions, w1_s, w1_v, w2_ss, w2_vv, W_out, b_out):
    sq = jnp.sum(positions * positions, axis=1)
    PE = jnp.concatenate(
        [positions,
         jnp.ones((N, 1), jnp.float32),
         sq[:, None],
         jnp.zeros((N, 3), jnp.float32)], axis=1)
    Pt = jnp.concatenate([positions.T, sq[None, :]], axis=0)
    ab = jnp.stack([jax.nn.gelu(w1_s) @ w2_ss, w1_v @ w2_vv], axis=0)
    out = pl.pallas_call(
        _body,
        out_shape=jax.ShapeDtypeStruct((1, 10), jnp.float32),
    )(PE, Pt, ab, W_out, b_out[None, :])
    return out[0]


# speedup vs baseline: 1.2477x; 1.2477x over previous
import functools

import jax
import jax.numpy as jnp
from jax.experimental import pallas as pl

N = 2048
BM = 256
NB = N // BM
CUTOFF2 = 1.5 * 1.5


def _w_block(jb, Ar, Bc):
    rows = slice(jb * BM, (jb + 1) * BM)
    d2 = jax.lax.dot_general(Ar[rows, :], Bc, (((1,), (0,)), ((), ())),
                             preferred_element_type=jnp.float32)
    row_ids = jax.lax.broadcasted_iota(jnp.int32, (BM, N), 0) + jb * BM
    col_ids = jax.lax.broadcasted_iota(jnp.int32, (BM, N), 1)
    mask = (d2 < CUTOFF2) & (row_ids != col_ids)
    W = jnp.where(mask, jax.lax.rsqrt(jnp.maximum(d2, 0.0) + 1e-12), 0.0)
    deg = jnp.sum(jnp.where(mask, 1.0, 0.0), axis=1, keepdims=True)
    invdeg = 1.0 / jnp.maximum(deg, 1.0)
    has = (deg > 0.0).astype(jnp.float32)
    return W, invdeg, has


def _body(ar_ref, bc_ref, ab_ref, wo_ref, bo_ref, out_ref):
    Ar = ar_ref[:]
    Bc = bc_ref[:]
    a = ab_ref[0:1, :]
    b = ab_ref[1:2, :]
    X4 = Ar[:, 0:4]

    uc_blocks = []
    aux = []
    for jb in range(NB):
        W, invdeg, has = _w_block(jb, Ar, Bc)
        Pr = X4[jb * BM:(jb + 1) * BM, 0:3]
        A = jnp.dot(W, X4, preferred_element_type=jnp.float32)
        U = (Pr * A[:, 3:4] - A[:, 0:3]) * invdeg
        c = jnp.sum(U * Pr, axis=1, keepdims=True)
        uc_blocks.append(jnp.concatenate([U, c], axis=1))
        aux.append((invdeg, has, Pr))
    UC = jnp.concatenate(uc_blocks, axis=0)

    acc = jnp.zeros((1, 10), dtype=jnp.float32)
    for jb in range(NB):
        W, invdeg, has = _w_block(jb, Ar, Bc)
        Pr = aux[jb][2]
        B = jnp.dot(W, UC, preferred_element_type=jnp.float32)
        t = (jnp.sum(Pr * B[:, 0:3], axis=1, keepdims=True)
             - B[:, 3:4]) * invdeg
        node2 = jax.nn.gelu(has * a + t * b)
        acc = acc + jnp.sum(node2, axis=0, keepdims=True)

    pooled = acc * (1.0 / N)
    logits = jnp.dot(pooled, wo_ref[:],
                     preferred_element_type=jnp.float32) + bo_ref[:]
    out_ref[:] = jax.nn.softmax(logits, axis=-1)


@functools.partial(jax.jit, static_argnames=())
def kernel(positions, w1_s, w1_v, w2_ss, w2_vv, W_out, b_out):
    sq = jnp.sum(positions * positions, axis=1)
    ones = jnp.ones((N, 1), jnp.float32)
    zeros = jnp.zeros((N, 3), jnp.float32)
    Ar = jnp.concatenate([positions, ones, sq[:, None], zeros], axis=1)
    Bc = jnp.concatenate([-2.0 * positions.T, sq[None, :],
                          jnp.ones((1, N), jnp.float32),
                          jnp.zeros((3, N), jnp.float32)], axis=0)
    ab = jnp.stack([jax.nn.gelu(w1_s) @ w2_ss, w1_v @ w2_vv], axis=0)
    out = pl.pallas_call(
        _body,
        out_shape=jax.ShapeDtypeStruct((1, 10), jnp.float32),
    )(Ar, Bc, ab, W_out, b_out[None, :])
    return out[0]
